# baseline (speedup 1.0000x reference)
import functools

import jax
import jax.numpy as jnp
from jax import lax
from jax.experimental import pallas as pl
from jax.experimental.pallas import tpu as pltpu
from jax.experimental.pallas import tpu_sc as plsc

_THR = 0.8
_N = 1024
_CH = 128
_LCH = 256
_NLC = _N // _LCH
_S = 32
_F = 5
_NC = 2
_NS = 16
_L = 16


def _keep_kernel(raw_ref, tr_ref, num_ref, keep_ref):
    n = num_ref[0, 0, 0]
    tr = tr_ref[0]
    icc = lax.broadcasted_iota(jnp.int32, (_CH, 1), 0)
    ilc = lax.broadcasted_iota(jnp.int32, (1, _LCH), 1)
    nch = (n + _CH - 1) // _CH

    for lc in range(_NLC):
        l0 = lc * _LCH

        @pl.when(n > l0)
        def _compute(lc=lc, l0=l0):
            x1r = tr[0:1, l0:l0 + _LCH]
            y1r = tr[1:2, l0:l0 + _LCH]
            x2r = tr[2:3, l0:l0 + _LCH]
            y2r = tr[3:4, l0:l0 + _LCH]
            area_r = (x2r - x1r) * (y2r - y1r)

            def body(ci, acc):
                a0 = ci * _CH
                ch = raw_ref[0, pl.ds(a0, _CH), :]
                vc = (icc + a0) < n
                x1c = jnp.where(vc, ch[:, 0:1], -jnp.inf)
                y1c = ch[:, 1:2]
                x2c, y2c = ch[:, 2:3], ch[:, 3:4]
                area_c = (x2c - ch[:, 0:1]) * (y2c - y1c)
                d = (x1c >= x1r) & (y1c >= y1r) & (x2c <= x2r) & (y2c <= y2r)
                w = jnp.where(d, jnp.broadcast_to(area_c, (_CH, _LCH)), 0.0)
                return acc + jnp.sum(w, axis=0, keepdims=True)

            sum_self = lax.fori_loop(0, nch, body,
                                     jnp.zeros((1, _LCH), jnp.float32))
            vr = (ilc + l0) < n
            keep = (sum_self <= (1.0 + _THR) * area_r + _THR * 1e-9) & vr
            keep_ref[0, 0:1, l0:l0 + _LCH] = keep.astype(jnp.int32)

        @pl.when(n <= l0)
        def _zero(l0=l0):
            keep_ref[0, 0:1, l0:l0 + _LCH] = jnp.zeros((1, _LCH), jnp.int32)


def _sc_compact_body(rows_hbm, keep_hbm, num_hbm, out_hbm, cnt_hbm,
                     keep_v, rows_v, out_v, num_v, cnt_v):
    wid = lax.axis_index("s") * _NC + lax.axis_index("c")
    pltpu.sync_copy(keep_hbm.at[wid], keep_v)
    pltpu.sync_copy(rows_hbm.at[wid], rows_v)
    pltpu.sync_copy(num_hbm.at[wid], num_v)
    n = num_v[...][0]

    li = lax.iota(jnp.int32, _L)
    dnums = lax.GatherDimensionNumbers(
        offset_dims=(), collapsed_slice_dims=(0,), start_index_map=(0,))

    def body(ci, base):
        k = keep_v[pl.ds(ci * _L, _L)]
        c = k
        for sh in (1, 2, 4, 8):
            idx = jnp.maximum(li - sh, 0)
            g = lax.gather(c, idx[:, None], dnums, slice_sizes=(1,),
                           mode=lax.GatherScatterMode.PROMISE_IN_BOUNDS)
            c = c + jnp.where(li >= sh, g, 0)
        ce = c - k
        for i in range(_L):
            row = rows_v[pl.ds((ci * _L + i) * _L, _L)]
            p = base + ce[i]
            out_v[pl.ds(p * _L, _L)] = row
        return base + c[_L - 1]

    nk = lax.fori_loop(0, (n + _L - 1) // _L, body, jnp.int32(0))

    zero = jnp.zeros((_L,), jnp.float32)

    def ztail(i, carry):
        out_v[pl.ds((nk + i) * _L, _L)] = zero
        return carry
    lax.fori_loop(0, _N - nk, ztail, 0)

    cnt_v[...] = jnp.full((_L,), nk, jnp.int32)
    lax.cond(nk > 0,
             lambda: pltpu.sync_copy(out_v, out_hbm.at[wid]),
             lambda: pltpu.sync_copy(rows_v, out_hbm.at[wid]))
    pltpu.sync_copy(cnt_v, cnt_hbm.at[wid])


_sc_compact = functools.partial(
    pl.kernel,
    mesh=plsc.VectorSubcoreMesh(core_axis_name="c", subcore_axis_name="s"),
    out_type=[
        jax.ShapeDtypeStruct((_S, _N * _L), jnp.float32),
        jax.ShapeDtypeStruct((_S, _L), jnp.int32),
    ],
    scratch_types=[
        pltpu.VMEM((_N,), jnp.int32),
        pltpu.VMEM((_N * _L,), jnp.float32),
        pltpu.VMEM((_N * _L,), jnp.float32),
        pltpu.VMEM((_L,), jnp.int32),
        pltpu.VMEM((_L,), jnp.int32),
    ],
)(_sc_compact_body)


def kernel(box_prompts, num_boxes):
    T, C, MAXB, F = box_prompts.shape
    raw = box_prompts.reshape(_S, MAXB, F)
    raw = jnp.pad(raw, ((0, 0), (0, _N - MAXB), (0, 0)))
    tr = raw.transpose(0, 2, 1)
    num = num_boxes.reshape(_S, 1, 1)
    keep = pl.pallas_call(
        _keep_kernel,
        grid=(_S,),
        in_specs=[
            pl.BlockSpec((1, _N, F), lambda i: (i, 0, 0)),
            pl.BlockSpec((1, F, _N), lambda i: (i, 0, 0)),
            pl.BlockSpec((1, 1, 1), lambda i: (i, 0, 0), memory_space=pltpu.SMEM),
        ],
        out_specs=pl.BlockSpec((1, 1, _N), lambda i: (i, 0, 0)),
        out_shape=jax.ShapeDtypeStruct((_S, 1, _N), jnp.int32),
        compiler_params=pltpu.CompilerParams(
            dimension_semantics=("parallel",)
        ),
    )(raw, tr, num)
    rows = jnp.pad(raw, ((0, 0), (0, 0), (0, _L - F))).reshape(_S, _N * _L)
    numv = jnp.broadcast_to(num_boxes.reshape(_S, 1), (_S, _L))
    out_rows, cnts = _sc_compact(rows, keep.reshape(_S, _N), numv)
    filtered = (out_rows.reshape(_S, _N, _L)[:, :MAXB, :F]
                .reshape(T, C, MAXB, F))
    return filtered, cnts[:, 0].reshape(T, C)


# speedup vs baseline: 1.3408x; 1.3408x over previous
import functools

import jax
import jax.numpy as jnp
from jax import lax
from jax.experimental import pallas as pl
from jax.experimental.pallas import tpu as pltpu
from jax.experimental.pallas import tpu_sc as plsc

_THR = 0.8
_N = 1024
_CH = 128
_LCH = 256
_NLC = _N // _LCH
_S = 32
_F = 5
_NC = 2
_NS = 16
_L = 16


def _keep_kernel(raw_ref, tr_ref, num_ref, keep_ref):
    n = num_ref[0, 0, 0]
    tr = tr_ref[0]
    x1r, y1r, x2r, y2r = tr[0:1, :], tr[1:2, :], tr[2:3, :], tr[3:4, :]
    area_r = (x2r - x1r) * (y2r - y1r)
    ir = lax.broadcasted_iota(jnp.int32, (1, _N), 1)
    vr = ir < n
    icc = lax.broadcasted_iota(jnp.int32, (_CH, 1), 0)

    def body(ci, acc):
        a0 = ci * _CH
        ch = raw_ref[0, pl.ds(a0, _CH), :]
        vc = (icc + a0) < n
        x1c = jnp.where(vc, ch[:, 0:1], -jnp.inf)
        y1c = ch[:, 1:2]
        x2c, y2c = ch[:, 2:3], ch[:, 3:4]
        area_c = (x2c - ch[:, 0:1]) * (y2c - y1c)
        d = (x1c >= x1r) & (y1c >= y1r) & (x2c <= x2r) & (y2c <= y2r)
        w = jnp.where(d, jnp.broadcast_to(area_c, (_CH, _N)), 0.0)
        return acc + jnp.sum(w, axis=0, keepdims=True)

    nch = (n + _CH - 1) // _CH
    sum_self = lax.fori_loop(0, nch, body, jnp.zeros((1, _N), jnp.float32))
    keep = (sum_self <= (1.0 + _THR) * area_r + _THR * 1e-9) & vr
    keep_ref[0] = keep.astype(jnp.int32)


def _sc_compact_body(rows_hbm, keep_hbm, num_hbm, out_hbm, cnt_hbm,
                     keep_v, rows_v, out_v, num_v, cnt_v):
    wid = lax.axis_index("s") * _NC + lax.axis_index("c")
    pltpu.sync_copy(keep_hbm.at[wid], keep_v)
    pltpu.sync_copy(rows_hbm.at[wid], rows_v)
    pltpu.sync_copy(num_hbm.at[wid], num_v)
    n = num_v[...][0]

    li = lax.iota(jnp.int32, _L)
    dnums = lax.GatherDimensionNumbers(
        offset_dims=(), collapsed_slice_dims=(0,), start_index_map=(0,))

    def body(ci, base):
        k = keep_v[pl.ds(ci * _L, _L)]
        c = k
        for sh in (1, 2, 4, 8):
            idx = jnp.maximum(li - sh, 0)
            g = lax.gather(c, idx[:, None], dnums, slice_sizes=(1,),
                           mode=lax.GatherScatterMode.PROMISE_IN_BOUNDS)
            c = c + jnp.where(li >= sh, g, 0)
        ce = c - k
        for i in range(_L):
            row = rows_v[pl.ds((ci * _L + i) * _L, _L)]
            p = base + ce[i]
            out_v[pl.ds(p * _L, _L)] = row
        return base + c[_L - 1]

    nk = lax.fori_loop(0, (n + _L - 1) // _L, body, jnp.int32(0))

    zero = jnp.zeros((_L,), jnp.float32)

    def ztail(i, carry):
        out_v[pl.ds((nk + i) * _L, _L)] = zero
        return carry
    lax.fori_loop(0, _N - nk, ztail, 0)

    cnt_v[...] = jnp.full((_L,), nk, jnp.int32)
    lax.cond(nk > 0,
             lambda: pltpu.sync_copy(out_v, out_hbm.at[wid]),
             lambda: pltpu.sync_copy(rows_v, out_hbm.at[wid]))
    pltpu.sync_copy(cnt_v, cnt_hbm.at[wid])


_sc_compact = functools.partial(
    pl.kernel,
    mesh=plsc.VectorSubcoreMesh(core_axis_name="c", subcore_axis_name="s"),
    out_type=[
        jax.ShapeDtypeStruct((_S, _N * _L), jnp.float32),
        jax.ShapeDtypeStruct((_S, _L), jnp.int32),
    ],
    scratch_types=[
        pltpu.VMEM((_N,), jnp.int32),
        pltpu.VMEM((_N * _L,), jnp.float32),
        pltpu.VMEM((_N * _L,), jnp.float32),
        pltpu.VMEM((_L,), jnp.int32),
        pltpu.VMEM((_L,), jnp.int32),
    ],
)(_sc_compact_body)


def kernel(box_prompts, num_boxes):
    T, C, MAXB, F = box_prompts.shape
    raw = box_prompts.reshape(_S, MAXB, F)
    raw = jnp.pad(raw, ((0, 0), (0, _N - MAXB), (0, 0)))
    tr = raw.transpose(0, 2, 1)
    num = num_boxes.reshape(_S, 1, 1)
    keep = pl.pallas_call(
        _keep_kernel,
        grid=(_S,),
        in_specs=[
            pl.BlockSpec((1, _N, F), lambda i: (i, 0, 0)),
            pl.BlockSpec((1, F, _N), lambda i: (i, 0, 0)),
            pl.BlockSpec((1, 1, 1), lambda i: (i, 0, 0), memory_space=pltpu.SMEM),
        ],
        out_specs=pl.BlockSpec((1, 1, _N), lambda i: (i, 0, 0)),
        out_shape=jax.ShapeDtypeStruct((_S, 1, _N), jnp.int32),
        compiler_params=pltpu.CompilerParams(
            dimension_semantics=("parallel",)
        ),
    )(raw, tr, num)
    rows = jnp.pad(raw, ((0, 0), (0, 0), (0, _L - F))).reshape(_S, _N * _L)
    numv = jnp.broadcast_to(num_boxes.reshape(_S, 1), (_S, _L))
    out_rows, cnts = _sc_compact(rows, keep.reshape(_S, _N), numv)
    filtered = (out_rows.reshape(_S, _N, _L)[:, :MAXB, :F]
                .reshape(T, C, MAXB, F))
    return filtered, cnts[:, 0].reshape(T, C)
